# baseline (speedup 1.0000x reference)
import functools

import jax
import jax.numpy as jnp
from jax.experimental import pallas as pl
from jax.experimental.pallas import tpu as pltpu

S, H = 2048, 1024
E, NG, TOPK = 8, 4, 2
I = 512
SH_I = 1024
RSF = 2.5
NEG = -1e30


def _moe_body(x_ref, gk_ref, eb_ref, eg_ref, eu_ref, ed_ref,
              shg_ref, shu_ref, shd_ref, out_ref, w_ref, acc_ref):
    e = pl.program_id(0)

    @pl.when(e == 0)
    def _gate():
        x = x_ref[...]
        logits = jnp.dot(x, gk_ref[...], preferred_element_type=jnp.float32)
        scores = jax.nn.sigmoid(logits)
        sfc = scores + eb_ref[...]
        lane = jax.lax.broadcasted_iota(jnp.int32, (S, E), 1)
        grp = lane // (E // NG)
        pairsum = jnp.zeros_like(sfc)
        for g in range(NG):
            sg = jnp.sum(jnp.where(grp == g, sfc, 0.0), axis=1, keepdims=True)
            pairsum = pairsum + jnp.where(grp == g, sg, 0.0)
        m1 = jnp.max(pairsum, axis=1, keepdims=True)
        g1 = jnp.min(jnp.where(pairsum == m1, grp, E), axis=1, keepdims=True)
        p2 = jnp.where(grp == g1, NEG, pairsum)
        m2 = jnp.max(p2, axis=1, keepdims=True)
        g2 = jnp.min(jnp.where(p2 == m2, grp, E), axis=1, keepdims=True)
        gmask = (grp == g1) | (grp == g2)
        ms = jnp.where(gmask, sfc, 0.0)
        t1 = jnp.max(ms, axis=1, keepdims=True)
        i1 = jnp.min(jnp.where(ms == t1, lane, E), axis=1, keepdims=True)
        ms2 = jnp.where(lane == i1, NEG, ms)
        t2 = jnp.max(ms2, axis=1, keepdims=True)
        i2 = jnp.min(jnp.where(ms2 == t2, lane, E), axis=1, keepdims=True)
        denom = t1 + t2 + 1e-20
        w_ref[...] = (jnp.where(lane == i1, t1, 0.0)
                      + jnp.where(lane == i2, t2, 0.0)) * (RSF / denom)

    NCHUNK = 4
    CS = S // NCHUNK

    @pl.when(e < E)
    def _expert():
        lane = jax.lax.broadcasted_iota(jnp.int32, (S, E), 1)
        w = jnp.sum(jnp.where(lane == e, w_ref[...], 0.0), axis=1, keepdims=True)
        wd = ed_ref[0].astype(jnp.bfloat16)
        for h in range(NCHUNK):
            lo, hi = h * CS, (h + 1) * CS
            xh = x_ref[lo:hi, :]
            g = jnp.dot(xh, eg_ref[0],
                        preferred_element_type=jnp.float32).astype(jnp.bfloat16)
            u = jnp.dot(xh, eu_ref[0],
                        preferred_element_type=jnp.float32).astype(jnp.bfloat16)
            a = jax.nn.silu(g) * u
            y = jnp.dot(a, wd, preferred_element_type=jnp.float32)
            wh = w[lo:hi, :]

            @pl.when(e == 0)
            def _():
                acc_ref[lo:hi, :] = wh * y

            @pl.when(e > 0)
            def _():
                acc_ref[lo:hi, :] = acc_ref[lo:hi, :] + wh * y

    @pl.when(e == E)
    def _shared():
        wd = shd_ref[...].astype(jnp.bfloat16)
        for h in range(NCHUNK):
            lo, hi = h * CS, (h + 1) * CS
            xh = x_ref[lo:hi, :]
            g = jnp.dot(xh, shg_ref[...],
                        preferred_element_type=jnp.float32).astype(jnp.bfloat16)
            u = jnp.dot(xh, shu_ref[...],
                        preferred_element_type=jnp.float32).astype(jnp.bfloat16)
            a = jax.nn.silu(g) * u
            y = jnp.dot(a, wd, preferred_element_type=jnp.float32)
            out_ref[lo:hi, :] = acc_ref[lo:hi, :] + y


@jax.jit
def _moe(x, gate_kernel, e_bias, expert_gate, expert_up, expert_down,
         sh_gate, sh_up, sh_down):
    const = lambda e: (0, 0)
    return pl.pallas_call(
        _moe_body,
        grid=(E + 1,),
        in_specs=[
            pl.BlockSpec((S, H), const),
            pl.BlockSpec((H, E), const),
            pl.BlockSpec((1, E), const),
            pl.BlockSpec((1, H, I), lambda e: (jnp.minimum(e, E - 1), 0, 0)),
            pl.BlockSpec((1, H, I), lambda e: (jnp.minimum(e, E - 1), 0, 0)),
            pl.BlockSpec((1, I, H), lambda e: (jnp.minimum(e, E - 1), 0, 0)),
            pl.BlockSpec((H, SH_I), const),
            pl.BlockSpec((H, SH_I), const),
            pl.BlockSpec((SH_I, H), const),
        ],
        out_specs=pl.BlockSpec((S, H), const),
        out_shape=jax.ShapeDtypeStruct((S, H), jnp.float32),
        scratch_shapes=[
            pltpu.VMEM((S, E), jnp.float32),
            pltpu.VMEM((S, H), jnp.float32),
        ],
        compiler_params=pltpu.CompilerParams(
            dimension_semantics=("arbitrary",),
            vmem_limit_bytes=100 * 1024 * 1024,
        ),
    )(x, gate_kernel, e_bias, expert_gate, expert_up, expert_down,
      sh_gate, sh_up, sh_down)


def kernel(hidden_states, gate_kernel, e_bias, expert_gate, expert_up,
           expert_down, sh_gate, sh_up, sh_down):
    b, s, h = hidden_states.shape
    x = hidden_states.reshape(s, h)
    y = _moe(x, gate_kernel, e_bias.reshape(1, E), expert_gate, expert_up,
             expert_down, sh_gate, sh_up, sh_down)
    return y.reshape(b, s, h)


# speedup vs baseline: 1.0761x; 1.0761x over previous
import functools

import jax
import jax.numpy as jnp
from jax.experimental import pallas as pl
from jax.experimental.pallas import tpu as pltpu

S, H = 2048, 1024
E, NG, TOPK = 8, 4, 2
I = 512
SH_I = 1024
RSF = 2.5
NEG = -1e30


def _moe_body(x_ref, gk_ref, eb_ref, eg_ref, eu_ref, ed_ref,
              shg_ref, shu_ref, shd_ref, out_ref, w_ref, acc_ref, gu_ref):
    e = pl.program_id(0)

    @pl.when(e == 0)
    def _gate():
        x = x_ref[...]
        logits = jnp.dot(x, gk_ref[...], preferred_element_type=jnp.float32)
        scores = jax.nn.sigmoid(logits)
        sfc = scores + eb_ref[...]
        lane = jax.lax.broadcasted_iota(jnp.int32, (S, E), 1)
        grp = lane // (E // NG)
        pairsum = jnp.zeros_like(sfc)
        for g in range(NG):
            sg = jnp.sum(jnp.where(grp == g, sfc, 0.0), axis=1, keepdims=True)
            pairsum = pairsum + jnp.where(grp == g, sg, 0.0)
        m1 = jnp.max(pairsum, axis=1, keepdims=True)
        g1 = jnp.min(jnp.where(pairsum == m1, grp, E), axis=1, keepdims=True)
        p2 = jnp.where(grp == g1, NEG, pairsum)
        m2 = jnp.max(p2, axis=1, keepdims=True)
        g2 = jnp.min(jnp.where(p2 == m2, grp, E), axis=1, keepdims=True)
        gmask = (grp == g1) | (grp == g2)
        ms = jnp.where(gmask, sfc, 0.0)
        t1 = jnp.max(ms, axis=1, keepdims=True)
        i1 = jnp.min(jnp.where(ms == t1, lane, E), axis=1, keepdims=True)
        ms2 = jnp.where(lane == i1, NEG, ms)
        t2 = jnp.max(ms2, axis=1, keepdims=True)
        i2 = jnp.min(jnp.where(ms2 == t2, lane, E), axis=1, keepdims=True)
        denom = t1 + t2 + 1e-20
        w_ref[...] = (jnp.where(lane == i1, t1, 0.0)
                      + jnp.where(lane == i2, t2, 0.0)) * (RSF / denom)


    @pl.when((e >= 1) & (e <= E))
    def _down_prev():
        pe = e - 1
        lane = jax.lax.broadcasted_iota(jnp.int32, (S, E), 1)
        w = jnp.sum(jnp.where(lane == pe, w_ref[...], 0.0), axis=1,
                    keepdims=True)
        a = jax.nn.silu(gu_ref[:, :I]) * gu_ref[:, I:]
        y = jnp.dot(a, ed_ref[0].astype(jnp.bfloat16),
                    preferred_element_type=jnp.float32)

        @pl.when(e == 1)
        def _():
            acc_ref[...] = w * y

        @pl.when(e > 1)
        def _():
            acc_ref[...] = acc_ref[...] + w * y

    @pl.when(e < E)
    def _gu():
        xb = x_ref[...]
        gu_ref[:, :I] = jnp.dot(
            xb, eg_ref[0], preferred_element_type=jnp.float32
        ).astype(jnp.bfloat16)
        gu_ref[:, I:] = jnp.dot(
            xb, eu_ref[0], preferred_element_type=jnp.float32
        ).astype(jnp.bfloat16)

    @pl.when(e == E)
    def _shared():
        xb = x_ref[...]
        g = jnp.dot(xb, shg_ref[...],
                    preferred_element_type=jnp.float32).astype(jnp.bfloat16)
        u = jnp.dot(xb, shu_ref[...],
                    preferred_element_type=jnp.float32).astype(jnp.bfloat16)
        a = jax.nn.silu(g) * u
        y = jnp.dot(a, shd_ref[...].astype(jnp.bfloat16),
                    preferred_element_type=jnp.float32)
        out_ref[...] = acc_ref[...] + y


@jax.jit
def _moe(x, gate_kernel, e_bias, expert_gate, expert_up, expert_down,
         sh_gate, sh_up, sh_down):
    const = lambda e: (0, 0)
    return pl.pallas_call(
        _moe_body,
        grid=(E + 1,),
        in_specs=[
            pl.BlockSpec((S, H), const),
            pl.BlockSpec((H, E), const),
            pl.BlockSpec((1, E), const),
            pl.BlockSpec((1, H, I), lambda e: (jnp.minimum(e, E - 1), 0, 0)),
            pl.BlockSpec((1, H, I), lambda e: (jnp.minimum(e, E - 1), 0, 0)),
            pl.BlockSpec((1, I, H), lambda e: (jnp.clip(e - 1, 0, E - 1), 0, 0)),
            pl.BlockSpec((H, SH_I), const),
            pl.BlockSpec((H, SH_I), const),
            pl.BlockSpec((SH_I, H), const),
        ],
        out_specs=pl.BlockSpec((S, H), const),
        out_shape=jax.ShapeDtypeStruct((S, H), jnp.float32),
        scratch_shapes=[
            pltpu.VMEM((S, E), jnp.float32),
            pltpu.VMEM((S, H), jnp.float32),
            pltpu.VMEM((S, 2 * I), jnp.bfloat16),
        ],
        compiler_params=pltpu.CompilerParams(
            dimension_semantics=("arbitrary",),
            vmem_limit_bytes=100 * 1024 * 1024,
        ),
    )(x, gate_kernel, e_bias, expert_gate, expert_up, expert_down,
      sh_gate, sh_up, sh_down)


def kernel(hidden_states, gate_kernel, e_bias, expert_gate, expert_up,
           expert_down, sh_gate, sh_up, sh_down):
    b, s, h = hidden_states.shape
    x = hidden_states.reshape(s, h)
    y = _moe(x, gate_kernel, e_bias.reshape(1, E), expert_gate, expert_up,
             expert_down, sh_gate, sh_up, sh_down)
    return y.reshape(b, s, h)
